# baseline (speedup 1.0000x reference)
import functools

import jax
import jax.numpy as jnp
from jax import lax
from jax.experimental import pallas as pl
from jax.experimental.pallas import tpu as pltpu
from jax.experimental.pallas import tpu_sc as plsc

N_NODES = 10000
N_EDGES = 320000
FEATS = 128

NC = 2
NS = 16
NW = NC * NS
CHUNK = 128
NCHUNK = 84
NSUPER = NCHUNK // 2
E_PAD = NW * NCHUNK * CHUNK
ACC_ROWS = 10040
RPT = 632
RPT_LAST = ACC_ROWS - 15 * RPT
ROW_BLK = 1000


def _matmul_body(h_ref, w_ref, norm_ref, out_ref):
    out_ref[...] = (
        jnp.dot(h_ref[...], w_ref[...], preferred_element_type=jnp.float32)
        * norm_ref[...]
    )


def _post_body(p_ref, norm_ref, bias_ref, out_ref):
    z = (p_ref[0] + p_ref[1]) * norm_ref[...] + bias_ref[...]
    out_ref[...] = jnp.maximum(z, 0.2 * z)


def _agg_body(m_hbm, src_hbm, dst_hbm, zeros_hbm, out_hbm,
              srcs0, srcs1, srcs2, dsts0, dsts1, buf0, buf1, buf2, acc,
              gs0, gs1, gs2, is0, is1, is2, id0, id1):
    c = lax.axis_index("c")
    s = lax.axis_index("s")
    wid = s * NC + c

    src_slots = (srcs0, srcs1, srcs2)
    dst_slots = (dsts0, dsts1)
    bufs = (buf0, buf1, buf2)
    gsems = (gs0, gs1, gs2)
    isems = (is0, is1, is2)
    dsems = (id0, id1)

    @pl.when(s < NS - 1)
    def _():
        pltpu.sync_copy(zeros_hbm, acc.at[pl.ds(s * RPT, RPT)])

    @pl.when(s == NS - 1)
    def _():
        pltpu.sync_copy(zeros_hbm.at[pl.ds(0, RPT_LAST)],
                        acc.at[pl.ds((NS - 1) * RPT, RPT_LAST)])

    plsc.subcore_barrier()

    def emit(S, k0, l, src_refill, dst_refill, fire_gather):
        if l % 2 == 0:
            if src_refill:
                sl = (2 + l // 2) % 3
                pltpu.async_copy(src_hbm.at[wid, S + 2 + l // 2],
                                 src_slots[sl], isems[sl])
            if dst_refill:
                dl = (1 + l // 2) % 2
                pltpu.async_copy(dst_hbm.at[wid, S + 1 + l // 2],
                                 dst_slots[dl], dsems[dl])
        if fire_gather:
            sf = (1 + l // 2) % 3
            if l % 2 == 0:
                pltpu.make_async_copy(
                    src_hbm.at[wid, k0], src_slots[sf], isems[sf]).wait()
            bg = (l + 2) % 3
            pltpu.async_copy(m_hbm.at[src_slots[sf].at[(l + 2) % 2]],
                             bufs[bg], gsems[bg])
        b = l % 3
        pltpu.make_async_copy(m_hbm.at[src_slots[0].at[0]],
                              bufs[b], gsems[b]).wait()
        dd = (l // 2) % 2
        if l % 2 == 0:
            pltpu.make_async_copy(
                dst_hbm.at[wid, k0], dst_slots[dd], dsems[dd]).wait()
        pltpu.sync_copy(bufs[b], acc.at[dst_slots[dd].at[l % 2]], add=True)

    pltpu.sync_copy(src_hbm.at[wid, 0], srcs0)
    pltpu.async_copy(src_hbm.at[wid, 1], srcs1, is1)
    pltpu.async_copy(src_hbm.at[wid, 2], srcs2, is2)
    pltpu.async_copy(dst_hbm.at[wid, 0], dsts0, id0)
    pltpu.async_copy(m_hbm.at[srcs0.at[0]], buf0, gs0)
    pltpu.async_copy(m_hbm.at[srcs0.at[1]], buf1, gs1)
    for l in range(12):
        emit(0, 0, l, l >= 2, True, True)

    def body(k):
        S = 6 * k
        for l in range(12):
            emit(S, S, l, True, True, True)

    pl.loop(1, 6)(body)

    for l in range(12):
        emit(36, 36, l, l <= 6, l <= 8, l <= 9)

    plsc.subcore_barrier()

    @pl.when(s < NS - 1)
    def _():
        pltpu.sync_copy(acc.at[pl.ds(s * RPT, RPT)],
                        out_hbm.at[c].at[pl.ds(s * RPT, RPT)])

    @pl.when(s == NS - 1)
    def _():
        pltpu.sync_copy(acc.at[pl.ds((NS - 1) * RPT, RPT_LAST)],
                        out_hbm.at[c].at[pl.ds((NS - 1) * RPT, RPT_LAST)])


def kernel(h, edge_index, W, bias, norm):
    m = pl.pallas_call(
        _matmul_body,
        grid=(N_NODES // ROW_BLK,),
        in_specs=[
            pl.BlockSpec((ROW_BLK, FEATS), lambda i: (i, 0)),
            pl.BlockSpec((FEATS, FEATS), lambda i: (0, 0)),
            pl.BlockSpec((ROW_BLK, 1), lambda i: (i, 0)),
        ],
        out_specs=pl.BlockSpec((ROW_BLK, FEATS), lambda i: (i, 0)),
        out_shape=jax.ShapeDtypeStruct((N_NODES, FEATS), jnp.float32),
    )(h, W, norm)

    pad = E_PAD - N_EDGES
    pad_ids = jnp.arange(pad, dtype=jnp.int32)
    src = jnp.concatenate([edge_index[0], pad_ids % N_NODES])
    dst = jnp.concatenate(
        [edge_index[1], N_NODES + pad_ids % (ACC_ROWS - N_NODES)]
    )
    src4 = src.reshape(NW, NSUPER, 2, CHUNK)
    dst4 = dst.reshape(NW, NSUPER, 2, CHUNK)
    zeros = jnp.zeros((RPT, FEATS), jnp.float32)

    agg = functools.partial(
        pl.kernel,
        out_type=jax.ShapeDtypeStruct((NC, ACC_ROWS, FEATS), jnp.float32),
        mesh=plsc.VectorSubcoreMesh(core_axis_name="c", subcore_axis_name="s"),
        scratch_types=[
            pltpu.VMEM((2, CHUNK), jnp.int32),
            pltpu.VMEM((2, CHUNK), jnp.int32),
            pltpu.VMEM((2, CHUNK), jnp.int32),
            pltpu.VMEM((2, CHUNK), jnp.int32),
            pltpu.VMEM((2, CHUNK), jnp.int32),
            pltpu.VMEM((CHUNK, FEATS), jnp.float32),
            pltpu.VMEM((CHUNK, FEATS), jnp.float32),
            pltpu.VMEM((CHUNK, FEATS), jnp.float32),
            pltpu.VMEM_SHARED((ACC_ROWS, FEATS), jnp.float32),
            pltpu.SemaphoreType.DMA,
            pltpu.SemaphoreType.DMA,
            pltpu.SemaphoreType.DMA,
            pltpu.SemaphoreType.DMA,
            pltpu.SemaphoreType.DMA,
            pltpu.SemaphoreType.DMA,
            pltpu.SemaphoreType.DMA,
            pltpu.SemaphoreType.DMA,
        ],
    )(_agg_body)
    partial = agg(m, src4, dst4, zeros)

    n = pl.pallas_call(
        _post_body,
        grid=(N_NODES // ROW_BLK,),
        in_specs=[
            pl.BlockSpec((NC, ROW_BLK, FEATS), lambda i: (0, i, 0)),
            pl.BlockSpec((ROW_BLK, 1), lambda i: (i, 0)),
            pl.BlockSpec((1, FEATS), lambda i: (0, 0)),
        ],
        out_specs=pl.BlockSpec((ROW_BLK, FEATS), lambda i: (i, 0)),
        out_shape=jax.ShapeDtypeStruct((N_NODES, FEATS), jnp.float32),
    )(partial, norm, bias.reshape(1, FEATS))
    return n
